# Initial kernel scaffold; baseline (speedup 1.0000x reference)
import jax
import jax.numpy as jnp
from jax.experimental import pallas as pl


def kernel(x, Wn, bn, att_W, att_b, read_W, read_b, h1_W, h1_b, h2_W, h2_b):
    raise NotImplementedError("write your pallas kernel here")


# speedup vs baseline: 1.1254x; 1.1254x over previous
import jax
import jax.numpy as jnp
from jax.experimental import pallas as pl
from jax.experimental.pallas import tpu as pltpu

_N, _F, _H, _L, _T = 100000, 128, 64, 4, 5
_BN = 4000
_STEPS = _N // _BN


def _fused_kernel(x_ref, Wn_ref, bn_ref, attW_ref, attb_ref,
                  readW_ref, readb_ref, h1W_ref, h1b_ref, h2W_ref, h2b_ref,
                  o0_ref, o1_ref, o2_ref, o3_ref, o4_ref, acc_ref):
    step = pl.program_id(0)

    @pl.when(step == 0)
    def _init():
        acc_ref[...] = jnp.zeros_like(acc_ref)

    h = jnp.dot(x_ref[...], Wn_ref[...],
                preferred_element_type=jnp.float32) + bn_ref[...]
    for i in range(_L):
        h = jnp.dot(h, attW_ref[i], preferred_element_type=jnp.float32)
        h = jnp.maximum(h + attb_ref[i:i + 1, :], 0.0)
        acc_ref[i:i + 1, :] += jnp.sum(h, axis=0, keepdims=True)

    @pl.when(step == _STEPS - 1)
    def _readout():
        pooled = acc_ref[...] * (1.0 / _N)
        gr = jnp.sum(readb_ref[...], axis=0, keepdims=True)
        for i in range(_L):
            gr = gr + jnp.dot(pooled[i:i + 1, :], readW_ref[i],
                              preferred_element_type=jnp.float32)
        outs = (o0_ref, o1_ref, o2_ref, o3_ref, o4_ref)
        for j in range(_T):
            z = jnp.dot(gr, h1W_ref[j], preferred_element_type=jnp.float32)
            z = jnp.maximum(z + h1b_ref[j:j + 1, :], 0.0)
            o = (jnp.sum(z * h2W_ref[j:j + 1, :], axis=1, keepdims=True)
                 + h2b_ref[0:1, j:j + 1])
            if j in (0, 3, 4):
                o = jax.nn.sigmoid(o)
            outs[j][...] = o


def kernel(x, Wn, bn, att_W, att_b, read_W, read_b, h1_W, h1_b, h2_W, h2_b):
    bn2 = bn.reshape(1, _H)
    h2_Ws = h2_W[:, :, 0]
    h2_bs = h2_b.reshape(1, _T)

    whole = lambda a: pl.BlockSpec(a.shape, lambda i: (0,) * a.ndim)
    out_shape = tuple(jax.ShapeDtypeStruct((1, 1), jnp.float32)
                      for _ in range(_T))
    out_specs = tuple(pl.BlockSpec((1, 1), lambda i: (0, 0))
                      for _ in range(_T))

    return pl.pallas_call(
        _fused_kernel,
        grid=(_STEPS,),
        in_specs=[
            pl.BlockSpec((_BN, _F), lambda i: (i, 0)),
            whole(Wn), whole(bn2), whole(att_W), whole(att_b),
            whole(read_W), whole(read_b), whole(h1_W), whole(h1_b),
            whole(h2_Ws), whole(h2_bs),
        ],
        out_specs=out_specs,
        out_shape=out_shape,
        scratch_shapes=[pltpu.VMEM((_L, _H), jnp.float32)],
        compiler_params=pltpu.CompilerParams(
            dimension_semantics=("arbitrary",)),
    )(x, Wn, bn2, att_W, att_b, read_W, read_b, h1_W, h1_b, h2_Ws, h2_bs)
